# Initial kernel scaffold; baseline (speedup 1.0000x reference)
import jax
import jax.numpy as jnp
from jax.experimental import pallas as pl


def kernel(x, edge_index, W1, b1, W2, b2):
    raise NotImplementedError("write your pallas kernel here")


# speedup vs baseline: 8.8272x; 8.8272x over previous
import functools

import jax
import jax.numpy as jnp
from jax import lax
from jax.experimental import pallas as pl
from jax.experimental.pallas import tpu as pltpu
from jax.experimental.pallas import tpu_sc as plsc

N = 10000
E = 160000
D = 256
H = 128
NC, NS = 2, 16
NW = NC * NS

CH = 128
DEG_CHUNKS = 40
AGG_CHUNKS = 80
PAD_E = NW * DEG_CHUNKS * CH
ACC_ROWS = 10240
ZROWS = ACC_ROWS // NS
OROWS = ACC_ROWS // NS
DEG_W = 16
BN = 1000

_MESH = plsc.VectorSubcoreMesh(
    core_axis_name="c", subcore_axis_name="s", num_cores=NC, num_subcores=NS
)


@functools.partial(
    pl.kernel,
    out_type=jax.ShapeDtypeStruct((NC, ACC_ROWS, DEG_W), jnp.float32),
    mesh=_MESH,
    scratch_types=[
        pltpu.VMEM((DEG_CHUNKS, CH), jnp.int32),
        pltpu.VMEM((CH, DEG_W), jnp.float32),
        pltpu.VMEM_SHARED((ACC_ROWS, DEG_W), jnp.float32),
    ],
    compiler_params=pltpu.CompilerParams(use_tc_tiling_on_sc=False),
)
def _deg_sc(dst_hbm, ones_hbm, zeros_hbm, out_hbm, idx_v, ones_v, acc_sh):
    c = lax.axis_index("c")
    s = lax.axis_index("s")
    wid = c * NS + s
    pltpu.sync_copy(ones_hbm, ones_v)
    pltpu.sync_copy(zeros_hbm, acc_sh.at[pl.ds(s * ZROWS, ZROWS)])
    pltpu.sync_copy(dst_hbm.at[wid], idx_v)
    plsc.subcore_barrier()

    def chunk(j, carry):
        pltpu.sync_copy(ones_v, acc_sh.at[idx_v.at[j]], add=True)
        return carry

    lax.fori_loop(0, DEG_CHUNKS, chunk, 0)
    plsc.subcore_barrier()
    base = s * OROWS
    pltpu.sync_copy(
        acc_sh.at[pl.ds(base, OROWS)], out_hbm.at[c].at[pl.ds(base, OROWS)]
    )


IB = 16
NBLK = AGG_CHUNKS // IB


@functools.partial(
    pl.kernel,
    out_type=jax.ShapeDtypeStruct((NC, ACC_ROWS, H), jnp.float32),
    mesh=_MESH,
    scratch_types=[
        pltpu.VMEM((2, IB, CH), jnp.int32),
        pltpu.VMEM((2, IB, CH), jnp.int32),
        pltpu.VMEM((CH, H), jnp.float32),
        pltpu.VMEM((CH, H), jnp.float32),
        pltpu.SemaphoreType.DMA,
        pltpu.SemaphoreType.DMA,
        pltpu.SemaphoreType.DMA,
        pltpu.VMEM_SHARED((ACC_ROWS, H), jnp.float32),
    ],
)
def _agg_sc(
    y_hbm, src_hbm, dst_hbm, zeros_hbm, out_hbm,
    src_v, dst_v, rows0_v, rows1_v, sem0, sem1, semi, acc_sh,
):
    c = lax.axis_index("c")
    s = lax.axis_index("s")
    pltpu.sync_copy(zeros_hbm, acc_sh.at[pl.ds(s * ZROWS, ZROWS)])
    pltpu.sync_copy(src_hbm.at[s, pl.ds(0, IB)], src_v.at[0])
    pltpu.sync_copy(dst_hbm.at[s, pl.ds(0, IB)], dst_v.at[0])
    pltpu.async_copy(src_hbm.at[s, pl.ds(IB, IB)], src_v.at[1], semi)
    pltpu.async_copy(dst_hbm.at[s, pl.ds(IB, IB)], dst_v.at[1], semi)
    plsc.subcore_barrier()

    pltpu.async_copy(y_hbm.at[c].at[src_v.at[0, 0]], rows0_v, sem0)
    pltpu.async_copy(y_hbm.at[c].at[src_v.at[0, 1]], rows1_v, sem1)

    bufs = ((rows0_v, sem0), (rows1_v, sem1))

    for nb in range(NBLK):
        p = nb % 2
        q = (nb + 1) % 2
        if nb + 1 < NBLK:
            pltpu.make_async_copy(
                src_hbm.at[s, pl.ds((nb + 1) * IB, IB)], src_v.at[q], semi
            ).wait()
            pltpu.make_async_copy(
                dst_hbm.at[s, pl.ds((nb + 1) * IB, IB)], dst_v.at[q], semi
            ).wait()

        def step(g, carry, p=p):
            for b, (rows_v, sem) in enumerate(bufs):
                j = 2 * g + b
                pltpu.make_async_copy(
                    y_hbm.at[c].at[src_v.at[p, j]], rows_v, sem
                ).wait()
                pltpu.sync_copy(rows_v, acc_sh.at[dst_v.at[p, j]], add=True)
                pltpu.async_copy(
                    y_hbm.at[c].at[src_v.at[p, j + 2]], rows_v, sem
                )
            return carry

        lax.fori_loop(0, (IB - 2) // 2, step, 0)
        for b, (rows_v, sem) in enumerate(bufs):
            j = IB - 2 + b
            pltpu.make_async_copy(
                y_hbm.at[c].at[src_v.at[p, j]], rows_v, sem
            ).wait()
            pltpu.sync_copy(rows_v, acc_sh.at[dst_v.at[p, j]], add=True)
            if nb + 1 < NBLK:
                pltpu.async_copy(y_hbm.at[c].at[src_v.at[q, b]], rows_v, sem)
        if nb + 2 < NBLK:
            pltpu.async_copy(
                src_hbm.at[s, pl.ds((nb + 2) * IB, IB)], src_v.at[p], semi
            )
            pltpu.async_copy(
                dst_hbm.at[s, pl.ds((nb + 2) * IB, IB)], dst_v.at[p], semi
            )

    plsc.subcore_barrier()
    base = s * OROWS
    pltpu.sync_copy(
        acc_sh.at[pl.ds(base, OROWS)], out_hbm.at[c].at[pl.ds(base, OROWS)]
    )


def _dis_of(deg_ref):
    deg = deg_ref[0, :, 0] + deg_ref[1, :, 0]
    return jax.lax.rsqrt(1.0 + deg)[:, None]


def _tc1_body(x_ref, w_ref, deg_ref, y_ref):
    dis = _dis_of(deg_ref)
    xw = jnp.dot(x_ref[...], w_ref[...], preferred_element_type=jnp.float32)
    y_ref[0] = dis * xw[:, :H]
    y_ref[1] = dis * xw[:, H:]


def _tc2_body(agg_ref, y_ref, deg_ref, w_ref, b_ref, y2_ref):
    dis = _dis_of(deg_ref)
    b = b_ref[...]
    h0 = jnp.maximum(dis * (agg_ref[0] + y_ref[0]) + b[:, :H], 0.0)
    h1 = jnp.maximum(dis * (agg_ref[1] + y_ref[1]) + b[:, H:], 0.0)
    h = jnp.concatenate([h0, h1], axis=1)
    xw = jnp.dot(h, w_ref[...], preferred_element_type=jnp.float32)
    y2_ref[0] = dis * xw[:, :H]
    y2_ref[1] = dis * xw[:, H:]


def _tc3_body(agg_ref, y_ref, deg_ref, b_ref, out_ref):
    dis = _dis_of(deg_ref)
    b = b_ref[...]
    h0 = jnp.maximum(dis * (agg_ref[0] + y_ref[0]) + b[:, :H], 0.0)
    h1 = jnp.maximum(dis * (agg_ref[1] + y_ref[1]) + b[:, H:], 0.0)
    out_ref[...] = jnp.concatenate([h0, h1], axis=1)


_deg_spec = pl.BlockSpec((NC, BN, DEG_W), lambda i: (0, i, 0))
_half_spec = pl.BlockSpec((NC, BN, H), lambda i: (0, i, 0))
_b_spec = pl.BlockSpec((1, D), lambda i: (0, 0))

_tc1 = pl.pallas_call(
    _tc1_body,
    grid=(N // BN,),
    in_specs=[
        pl.BlockSpec((BN, D), lambda i: (i, 0)),
        pl.BlockSpec((D, D), lambda i: (0, 0)),
        _deg_spec,
    ],
    out_specs=_half_spec,
    out_shape=jax.ShapeDtypeStruct((NC, ACC_ROWS, H), jnp.float32),
)

_tc2 = pl.pallas_call(
    _tc2_body,
    grid=(N // BN,),
    in_specs=[
        _half_spec,
        _half_spec,
        _deg_spec,
        pl.BlockSpec((D, D), lambda i: (0, 0)),
        _b_spec,
    ],
    out_specs=_half_spec,
    out_shape=jax.ShapeDtypeStruct((NC, ACC_ROWS, H), jnp.float32),
)

_tc3 = pl.pallas_call(
    _tc3_body,
    grid=(N // BN,),
    in_specs=[_half_spec, _half_spec, _deg_spec, _b_spec],
    out_specs=pl.BlockSpec((BN, D), lambda i: (i, 0)),
    out_shape=jax.ShapeDtypeStruct((N, D), jnp.float32),
)


def kernel(x, edge_index, W1, b1, W2, b2):
    src = edge_index[0].astype(jnp.int32)
    dst = edge_index[1].astype(jnp.int32)
    pad = PAD_E - E
    src_pad = jnp.concatenate([src, jnp.zeros((pad,), jnp.int32)])
    dst_pad = jnp.concatenate([dst, jnp.full((pad,), N, jnp.int32)])
    dst_deg = dst_pad.reshape(NW, DEG_CHUNKS, CH)
    src_agg = src_pad.reshape(NS, AGG_CHUNKS, CH)
    dst_agg = dst_pad.reshape(NS, AGG_CHUNKS, CH)

    ones16 = jnp.ones((CH, DEG_W), jnp.float32)
    zeros16 = jnp.zeros((ZROWS, DEG_W), jnp.float32)
    zeros_h = jnp.zeros((ZROWS, H), jnp.float32)

    deg = _deg_sc(dst_deg, ones16, zeros16)
    y1 = _tc1(x, W1, deg)
    agg1 = _agg_sc(y1, src_agg, dst_agg, zeros_h)
    y2 = _tc2(agg1, y1, deg, W2, b1.reshape(1, D))
    agg2 = _agg_sc(y2, src_agg, dst_agg, zeros_h)
    return _tc3(agg2, y2, deg, b2.reshape(1, D))
